# baseline (speedup 1.0000x reference)
import jax
import jax.numpy as jnp
import numpy as np
from jax import lax
from jax.experimental import pallas as pl
from jax.experimental.pallas import tpu as pltpu
from jax.experimental.pallas import tpu_sc as plsc

N = 10000
E = 320000
F = 128
H = 64
O = 32

NC, NS = 2, 16
NW = NC * NS
NU = E // 256
UT = NU // NW
ZR = N // NS
RB = 1000
PB = 1000
W1 = H + 8


def _p1_body(x_ref, wl_ref, p_ref):
    mm = jnp.dot(x_ref[...], wl_ref[...], preferred_element_type=jnp.float32)
    p_ref[...] = jnp.concatenate(
        [mm, jnp.ones((PB, 1), jnp.float32), jnp.zeros((PB, 7), jnp.float32)],
        axis=1)


def _project_p1(x, wlT):
    return pl.pallas_call(
        _p1_body,
        grid=(N // PB,),
        in_specs=[
            pl.BlockSpec((PB, F), lambda i: (i, 0)),
            pl.BlockSpec((F, H), lambda i: (0, 0)),
        ],
        out_specs=pl.BlockSpec((PB, W1), lambda i: (i, 0)),
        out_shape=jax.ShapeDtypeStruct((N, W1), jnp.float32),
    )(x, wlT)


def _root_body(x_ref, w_ref, b_ref, r_ref):
    r_ref[...] = (jnp.dot(x_ref[...], w_ref[...],
                          preferred_element_type=jnp.float32) + b_ref[...])


def _project_root(x, wT, b, d_in, d_out):
    return pl.pallas_call(
        _root_body,
        grid=(N // PB,),
        in_specs=[
            pl.BlockSpec((PB, d_in), lambda i: (i, 0)),
            pl.BlockSpec((d_in, d_out), lambda i: (0, 0)),
            pl.BlockSpec((1, d_out), lambda i: (0, 0)),
        ],
        out_specs=pl.BlockSpec((PB, d_out), lambda i: (i, 0)),
        out_shape=jax.ShapeDtypeStruct((N, d_out), jnp.float32),
    )(x, wT, b.reshape(1, d_out))


def _combine_body(parts_ref, r_ref, wl_ref, p_ref, h_ref, ic_ref):
    agg = parts_ref[0, :, :H] + parts_ref[1, :, :H]
    cnt = parts_ref[0, :, H:H + 1] + parts_ref[1, :, H:H + 1]
    invc = 1.0 / jnp.maximum(cnt, 1.0)
    h = jnp.maximum(agg * invc + r_ref[...], 0.0)
    h_ref[...] = h
    ic_ref[...] = jnp.broadcast_to(invc, (RB, O))
    p_ref[...] = jnp.dot(h, wl_ref[...], preferred_element_type=jnp.float32)


def _combine_project(parts, r, wlT):
    return pl.pallas_call(
        _combine_body,
        grid=(N // RB,),
        in_specs=[
            pl.BlockSpec((NC, RB, 128), lambda i: (0, i, 0)),
            pl.BlockSpec((RB, H), lambda i: (i, 0)),
            pl.BlockSpec((H, O), lambda i: (0, 0)),
        ],
        out_specs=[
            pl.BlockSpec((RB, O), lambda i: (i, 0)),
            pl.BlockSpec((RB, H), lambda i: (i, 0)),
            pl.BlockSpec((RB, O), lambda i: (i, 0)),
        ],
        out_shape=[
            jax.ShapeDtypeStruct((N, O), jnp.float32),
            jax.ShapeDtypeStruct((N, H), jnp.float32),
            jax.ShapeDtypeStruct((N, O), jnp.float32),
        ],
    )(parts, r, wlT)


def _final_body(parts_ref, ic_ref, r_ref, out_ref):
    agg = parts_ref[0, :, :O] + parts_ref[1, :, :O]
    out_ref[...] = agg * ic_ref[...] + r_ref[...]


def _final_combine(parts, invc, r):
    return pl.pallas_call(
        _final_body,
        grid=(N // RB,),
        in_specs=[
            pl.BlockSpec((NC, RB, 128), lambda i: (0, i, 0)),
            pl.BlockSpec((RB, O), lambda i: (i, 0)),
            pl.BlockSpec((RB, O), lambda i: (i, 0)),
        ],
        out_specs=pl.BlockSpec((RB, O), lambda i: (i, 0)),
        out_shape=jax.ShapeDtypeStruct((N, O), jnp.float32),
    )(parts, invc, r)


def _make_edge_scatter(d):
    mesh = plsc.VectorSubcoreMesh(core_axis_name="c", subcore_axis_name="s",
                                  num_cores=NC, num_subcores=NS)

    def body(table, e3d, zeros_d, out, idx_v, rows0, rows1, rows2, acc,
             sem0, sem1, sem2):
        cid = lax.axis_index("c")
        sid = lax.axis_index("s")
        tile = cid * NS + sid
        extra = tile < NU - UT * NW
        base = tile * UT

        pltpu.sync_copy(e3d.at[:, pl.ds(base, UT)], idx_v.at[:, pl.ds(0, UT)])

        @pl.when(extra)
        def _():
            pltpu.sync_copy(e3d.at[:, pl.ds(UT * NW + tile, 1)],
                            idx_v.at[:, pl.ds(UT, 1)])

        pltpu.async_copy(table.at[idx_v.at[0, 0]], rows0, sem0)
        pltpu.async_copy(table.at[idx_v.at[0, 1]], rows1, sem1)

        zlo = sid * ZR
        pltpu.sync_copy(zeros_d.at[pl.ds(zlo, ZR)], acc.at[pl.ds(zlo, ZR)])
        plsc.subcore_barrier()

        nt = UT // 3

        def triple(k, carry):
            u = 3 * k
            pltpu.async_copy(table.at[idx_v.at[0, u + 2]], rows2, sem2)
            pltpu.make_async_copy(table.at[idx_v.at[0, u]], rows0,
                                  sem0).wait()
            pltpu.sync_copy(rows0, acc.at[idx_v.at[1, u]], add=True)

            @pl.when(k < nt - 1)
            def _():
                pltpu.async_copy(table.at[idx_v.at[0, u + 3]], rows0, sem0)

            pltpu.make_async_copy(table.at[idx_v.at[0, u + 1]], rows1,
                                  sem1).wait()
            pltpu.sync_copy(rows1, acc.at[idx_v.at[1, u + 1]], add=True)

            @pl.when(k < nt - 1)
            def _():
                pltpu.async_copy(table.at[idx_v.at[0, u + 4]], rows1, sem1)

            pltpu.make_async_copy(table.at[idx_v.at[0, u + 2]], rows2,
                                  sem2).wait()
            pltpu.sync_copy(rows2, acc.at[idx_v.at[1, u + 2]], add=True)
            return carry

        lax.fori_loop(0, nt, triple, None)

        @pl.when(extra)
        def _():
            pltpu.async_copy(table.at[idx_v.at[0, UT]], rows0, sem0)
            pltpu.make_async_copy(table.at[idx_v.at[0, UT]], rows0,
                                  sem0).wait()
            pltpu.sync_copy(rows0, acc.at[idx_v.at[1, UT]], add=True)

        plsc.subcore_barrier()

        pltpu.sync_copy(acc.at[pl.ds(zlo, ZR)],
                        out.at[cid, pl.ds(zlo, ZR), pl.ds(0, d)])

    return pl.kernel(
        body,
        out_type=[jax.ShapeDtypeStruct((NC, N, 128), jnp.float32)],
        mesh=mesh,
        scratch_types=[
            pltpu.VMEM((2, UT + 1, 256), jnp.int32),
            pltpu.VMEM((256, d), jnp.float32),
            pltpu.VMEM((256, d), jnp.float32),
            pltpu.VMEM((256, d), jnp.float32),
            pltpu.VMEM_SHARED((N, d), jnp.float32),
            pltpu.SemaphoreType.DMA,
            pltpu.SemaphoreType.DMA,
            pltpu.SemaphoreType.DMA,
        ],
        compiler_params=pltpu.CompilerParams(use_tc_tiling_on_sc=False))


_edge_scatter_l1 = _make_edge_scatter(W1)
_edge_scatter_l2 = _make_edge_scatter(O)


_ZEROS_1 = np.zeros((N, W1), np.float32)
_ZEROS_1[0, W1 - 1] = 1e-30
_ZEROS_2 = np.zeros((N, O), np.float32)
_ZEROS_2[0, 0] = 1e-30


def kernel(x, edge_index, W1l, b1, W1r, W2l, b2, W2r):
    e3d = edge_index.astype(jnp.int32).reshape(2, NU, 256)
    zeros_1 = _ZEROS_1
    zeros_2 = _ZEROS_2

    p1 = _project_p1(x, W1l.T)
    part1, = _edge_scatter_l1(p1, e3d, zeros_1)
    r1 = _project_root(x, W1r.T, b1, F, H)
    p2, h, invc = _combine_project(part1, r1, W2l.T)
    part2, = _edge_scatter_l2(p2, e3d, zeros_2)
    r2 = _project_root(h, W2r.T, b2, H, O)
    return _final_combine(part2, invc, r2)


# speedup vs baseline: 1.0380x; 1.0380x over previous
import jax
import jax.numpy as jnp
import numpy as np
from jax import lax
from jax.experimental import pallas as pl
from jax.experimental.pallas import tpu as pltpu
from jax.experimental.pallas import tpu_sc as plsc

N = 10000
E = 320000
F = 128
H = 64
O = 32

NC, NS = 2, 16
NW = NC * NS
NU = E // 256
UT = NU // NW
ZR = N // NS
RB = 1000
W1 = H + 8


def _p1_body(x_ref, wl_ref, p_ref):
    mm = jnp.dot(x_ref[...], wl_ref[...], preferred_element_type=jnp.float32)
    p_ref[...] = jnp.concatenate(
        [mm, jnp.ones((RB, 1), jnp.float32), jnp.zeros((RB, 7), jnp.float32)],
        axis=1)


def _project_p1(x, wlT):
    return pl.pallas_call(
        _p1_body,
        grid=(N // RB,),
        in_specs=[
            pl.BlockSpec((RB, F), lambda i: (i, 0)),
            pl.BlockSpec((F, H), lambda i: (0, 0)),
        ],
        out_specs=pl.BlockSpec((RB, W1), lambda i: (i, 0)),
        out_shape=jax.ShapeDtypeStruct((N, W1), jnp.float32),
    )(x, wlT)


def _root_body(x_ref, w_ref, b_ref, r_ref):
    r_ref[...] = (jnp.dot(x_ref[...], w_ref[...],
                          preferred_element_type=jnp.float32) + b_ref[...])


def _project_root(x, wT, b, d_in, d_out):
    return pl.pallas_call(
        _root_body,
        grid=(N // RB,),
        in_specs=[
            pl.BlockSpec((RB, d_in), lambda i: (i, 0)),
            pl.BlockSpec((d_in, d_out), lambda i: (0, 0)),
            pl.BlockSpec((1, d_out), lambda i: (0, 0)),
        ],
        out_specs=pl.BlockSpec((RB, d_out), lambda i: (i, 0)),
        out_shape=jax.ShapeDtypeStruct((N, d_out), jnp.float32),
    )(x, wT, b.reshape(1, d_out))


def _combine_body(parts_ref, r_ref, wl_ref, p_ref, h_ref, ic_ref):
    agg = parts_ref[0, :, :H] + parts_ref[1, :, :H]
    cnt = parts_ref[0, :, H:H + 1] + parts_ref[1, :, H:H + 1]
    invc = 1.0 / jnp.maximum(cnt, 1.0)
    h = jnp.maximum(agg * invc + r_ref[...], 0.0)
    h_ref[...] = h
    ic_ref[...] = jnp.broadcast_to(invc, (RB, O))
    p_ref[...] = jnp.dot(h, wl_ref[...], preferred_element_type=jnp.float32)


def _combine_project(parts, r, wlT):
    return pl.pallas_call(
        _combine_body,
        grid=(N // RB,),
        in_specs=[
            pl.BlockSpec((NC, RB, 128), lambda i: (0, i, 0)),
            pl.BlockSpec((RB, H), lambda i: (i, 0)),
            pl.BlockSpec((H, O), lambda i: (0, 0)),
        ],
        out_specs=[
            pl.BlockSpec((RB, O), lambda i: (i, 0)),
            pl.BlockSpec((RB, H), lambda i: (i, 0)),
            pl.BlockSpec((RB, O), lambda i: (i, 0)),
        ],
        out_shape=[
            jax.ShapeDtypeStruct((N, O), jnp.float32),
            jax.ShapeDtypeStruct((N, H), jnp.float32),
            jax.ShapeDtypeStruct((N, O), jnp.float32),
        ],
    )(parts, r, wlT)


def _final_body(parts_ref, ic_ref, r_ref, out_ref):
    agg = parts_ref[0, :, :O] + parts_ref[1, :, :O]
    out_ref[...] = agg * ic_ref[...] + r_ref[...]


def _final_combine(parts, invc, r):
    return pl.pallas_call(
        _final_body,
        grid=(N // RB,),
        in_specs=[
            pl.BlockSpec((NC, RB, 128), lambda i: (0, i, 0)),
            pl.BlockSpec((RB, O), lambda i: (i, 0)),
            pl.BlockSpec((RB, O), lambda i: (i, 0)),
        ],
        out_specs=pl.BlockSpec((RB, O), lambda i: (i, 0)),
        out_shape=jax.ShapeDtypeStruct((N, O), jnp.float32),
    )(parts, invc, r)


def _make_edge_scatter(d):
    mesh = plsc.VectorSubcoreMesh(core_axis_name="c", subcore_axis_name="s",
                                  num_cores=NC, num_subcores=NS)

    def body(table, e3d, zeros_d, out, idx_v, rows0, rows1, rows2, acc,
             sem0, sem1, sem2):
        cid = lax.axis_index("c")
        sid = lax.axis_index("s")
        tile = cid * NS + sid
        extra = tile < NU - UT * NW
        base = tile * UT

        pltpu.sync_copy(e3d.at[:, pl.ds(base, UT)], idx_v.at[:, pl.ds(0, UT)])

        @pl.when(extra)
        def _():
            pltpu.sync_copy(e3d.at[:, pl.ds(UT * NW + tile, 1)],
                            idx_v.at[:, pl.ds(UT, 1)])

        pltpu.async_copy(table.at[idx_v.at[0, 0]], rows0, sem0)
        pltpu.async_copy(table.at[idx_v.at[0, 1]], rows1, sem1)

        zlo = sid * ZR
        pltpu.sync_copy(zeros_d.at[pl.ds(zlo, ZR)], acc.at[pl.ds(zlo, ZR)])
        plsc.subcore_barrier()

        nt = UT // 3

        def triple(k, carry):
            u = 3 * k
            pltpu.async_copy(table.at[idx_v.at[0, u + 2]], rows2, sem2)
            pltpu.make_async_copy(table.at[idx_v.at[0, u]], rows0,
                                  sem0).wait()
            pltpu.sync_copy(rows0, acc.at[idx_v.at[1, u]], add=True)

            @pl.when(k < nt - 1)
            def _():
                pltpu.async_copy(table.at[idx_v.at[0, u + 3]], rows0, sem0)

            pltpu.make_async_copy(table.at[idx_v.at[0, u + 1]], rows1,
                                  sem1).wait()
            pltpu.sync_copy(rows1, acc.at[idx_v.at[1, u + 1]], add=True)

            @pl.when(k < nt - 1)
            def _():
                pltpu.async_copy(table.at[idx_v.at[0, u + 4]], rows1, sem1)

            pltpu.make_async_copy(table.at[idx_v.at[0, u + 2]], rows2,
                                  sem2).wait()
            pltpu.sync_copy(rows2, acc.at[idx_v.at[1, u + 2]], add=True)
            return carry

        lax.fori_loop(0, nt, triple, None)

        @pl.when(extra)
        def _():
            pltpu.async_copy(table.at[idx_v.at[0, UT]], rows0, sem0)
            pltpu.make_async_copy(table.at[idx_v.at[0, UT]], rows0,
                                  sem0).wait()
            pltpu.sync_copy(rows0, acc.at[idx_v.at[1, UT]], add=True)

        plsc.subcore_barrier()

        pltpu.sync_copy(acc.at[pl.ds(zlo, ZR)],
                        out.at[cid, pl.ds(zlo, ZR), pl.ds(0, d)])

    return pl.kernel(
        body,
        out_type=[jax.ShapeDtypeStruct((NC, N, 128), jnp.float32)],
        mesh=mesh,
        scratch_types=[
            pltpu.VMEM((2, UT + 1, 256), jnp.int32),
            pltpu.VMEM((256, d), jnp.float32),
            pltpu.VMEM((256, d), jnp.float32),
            pltpu.VMEM((256, d), jnp.float32),
            pltpu.VMEM_SHARED((N, d), jnp.float32),
            pltpu.SemaphoreType.DMA,
            pltpu.SemaphoreType.DMA,
            pltpu.SemaphoreType.DMA,
        ],
        compiler_params=pltpu.CompilerParams(use_tc_tiling_on_sc=False))


_edge_scatter_l1 = _make_edge_scatter(W1)
_edge_scatter_l2 = _make_edge_scatter(O)


_ZEROS_1 = np.zeros((N, W1), np.float32)
_ZEROS_2 = np.zeros((N, O), np.float32)


def kernel(x, edge_index, W1l, b1, W1r, W2l, b2, W2r):
    e3d = edge_index.astype(jnp.int32).reshape(2, NU, 256)
    zeros_1 = _ZEROS_1
    zeros_2 = _ZEROS_2

    p1 = _project_p1(x, W1l.T)
    part1, = _edge_scatter_l1(p1, e3d, zeros_1)
    r1 = _project_root(x, W1r.T, b1, F, H)
    p2, h, invc = _combine_project(part1, r1, W2l.T)
    part2, = _edge_scatter_l2(p2, e3d, zeros_2)
    r2 = _project_root(h, W2r.T, b2, H, O)
    return _final_combine(part2, invc, r2)


# speedup vs baseline: 1.0409x; 1.0028x over previous
import jax
import jax.numpy as jnp
import numpy as np
from jax import lax
from jax.experimental import pallas as pl
from jax.experimental.pallas import tpu as pltpu
from jax.experimental.pallas import tpu_sc as plsc

N = 10000
E = 320000
F = 128
H = 64
O = 32

NC, NS = 2, 16
NW = NC * NS
NU = E // 256
UT = NU // NW
ZR = N // NS
RB = 1000
W1 = H + 8


def _p1_body(x_ref, wl_ref, p_ref):
    mm = jnp.dot(x_ref[...], wl_ref[...], preferred_element_type=jnp.float32)
    p_ref[...] = jnp.concatenate(
        [mm, jnp.ones((RB, 1), jnp.float32), jnp.zeros((RB, 7), jnp.float32)],
        axis=1)


def _project_p1(x, wlT):
    return pl.pallas_call(
        _p1_body,
        grid=(N // RB,),
        in_specs=[
            pl.BlockSpec((RB, F), lambda i: (i, 0)),
            pl.BlockSpec((F, H), lambda i: (0, 0)),
        ],
        out_specs=pl.BlockSpec((RB, W1), lambda i: (i, 0)),
        out_shape=jax.ShapeDtypeStruct((N, W1), jnp.float32),
    )(x, wlT)


def _root_body(x_ref, w_ref, b_ref, r_ref):
    r_ref[...] = (jnp.dot(x_ref[...], w_ref[...],
                          preferred_element_type=jnp.float32) + b_ref[...])


def _project_root(x, wT, b, d_in, d_out):
    return pl.pallas_call(
        _root_body,
        grid=(N // RB,),
        in_specs=[
            pl.BlockSpec((RB, d_in), lambda i: (i, 0)),
            pl.BlockSpec((d_in, d_out), lambda i: (0, 0)),
            pl.BlockSpec((1, d_out), lambda i: (0, 0)),
        ],
        out_specs=pl.BlockSpec((RB, d_out), lambda i: (i, 0)),
        out_shape=jax.ShapeDtypeStruct((N, d_out), jnp.float32),
    )(x, wT, b.reshape(1, d_out))


def _combine_body(parts_ref, r_ref, wl_ref, p_ref, h_ref, ic_ref):
    agg = parts_ref[0, :, :H] + parts_ref[1, :, :H]
    cnt = parts_ref[0, :, H:H + 1] + parts_ref[1, :, H:H + 1]
    invc = 1.0 / jnp.maximum(cnt, 1.0)
    h = jnp.maximum(agg * invc + r_ref[...], 0.0)
    h_ref[...] = h
    ic_ref[...] = jnp.broadcast_to(invc, (RB, O))
    p_ref[...] = jnp.dot(h, wl_ref[...], preferred_element_type=jnp.float32)


def _combine_project(parts, r, wlT):
    return pl.pallas_call(
        _combine_body,
        grid=(N // RB,),
        in_specs=[
            pl.BlockSpec((NC, RB, 128), lambda i: (0, i, 0)),
            pl.BlockSpec((RB, H), lambda i: (i, 0)),
            pl.BlockSpec((H, O), lambda i: (0, 0)),
        ],
        out_specs=[
            pl.BlockSpec((RB, O), lambda i: (i, 0)),
            pl.BlockSpec((RB, H), lambda i: (i, 0)),
            pl.BlockSpec((RB, O), lambda i: (i, 0)),
        ],
        out_shape=[
            jax.ShapeDtypeStruct((N, O), jnp.float32),
            jax.ShapeDtypeStruct((N, H), jnp.float32),
            jax.ShapeDtypeStruct((N, O), jnp.float32),
        ],
    )(parts, r, wlT)


def _final_body(parts_ref, ic_ref, r_ref, out_ref):
    agg = parts_ref[0, :, :O] + parts_ref[1, :, :O]
    out_ref[...] = agg * ic_ref[...] + r_ref[...]


def _final_combine(parts, invc, r):
    return pl.pallas_call(
        _final_body,
        grid=(N // RB,),
        in_specs=[
            pl.BlockSpec((NC, RB, 128), lambda i: (0, i, 0)),
            pl.BlockSpec((RB, O), lambda i: (i, 0)),
            pl.BlockSpec((RB, O), lambda i: (i, 0)),
        ],
        out_specs=pl.BlockSpec((RB, O), lambda i: (i, 0)),
        out_shape=jax.ShapeDtypeStruct((N, O), jnp.float32),
    )(parts, invc, r)


def _make_edge_scatter(d):
    mesh = plsc.VectorSubcoreMesh(core_axis_name="c", subcore_axis_name="s",
                                  num_cores=NC, num_subcores=NS)

    def body(table, e3d, zeros_d, out, idx_v, rows0, rows1, rows2, acc,
             sem0, sem1, sem2):
        cid = lax.axis_index("c")
        sid = lax.axis_index("s")
        tile = cid * NS + sid
        extra = tile < NU - UT * NW
        base = tile * UT

        pltpu.sync_copy(e3d.at[:, pl.ds(base, UT)], idx_v.at[:, pl.ds(0, UT)])

        @pl.when(extra)
        def _():
            pltpu.sync_copy(e3d.at[:, pl.ds(UT * NW + tile, 1)],
                            idx_v.at[:, pl.ds(UT, 1)])

        pltpu.async_copy(table.at[idx_v.at[0, 0]], rows0, sem0)
        pltpu.async_copy(table.at[idx_v.at[0, 1]], rows1, sem1)

        zlo = sid * ZR
        pltpu.sync_copy(zeros_d.at[pl.ds(zlo, ZR)], acc.at[pl.ds(zlo, ZR)])
        plsc.subcore_barrier()

        nt = UT // 3

        def triple(k, carry):
            u = 3 * k
            pltpu.async_copy(table.at[idx_v.at[0, u + 2]], rows2, sem2)
            pltpu.make_async_copy(table.at[idx_v.at[0, u]], rows0,
                                  sem0).wait()
            pltpu.sync_copy(rows0, acc.at[idx_v.at[1, u]], add=True)

            @pl.when(k < nt - 1)
            def _():
                pltpu.async_copy(table.at[idx_v.at[0, u + 3]], rows0, sem0)

            pltpu.make_async_copy(table.at[idx_v.at[0, u + 1]], rows1,
                                  sem1).wait()
            pltpu.sync_copy(rows1, acc.at[idx_v.at[1, u + 1]], add=True)

            @pl.when(k < nt - 1)
            def _():
                pltpu.async_copy(table.at[idx_v.at[0, u + 4]], rows1, sem1)

            pltpu.make_async_copy(table.at[idx_v.at[0, u + 2]], rows2,
                                  sem2).wait()
            pltpu.sync_copy(rows2, acc.at[idx_v.at[1, u + 2]], add=True)
            return carry

        lax.fori_loop(0, nt, triple, None)

        @pl.when(extra)
        def _():
            pltpu.async_copy(table.at[idx_v.at[0, UT]], rows0, sem0)
            pltpu.make_async_copy(table.at[idx_v.at[0, UT]], rows0,
                                  sem0).wait()
            pltpu.sync_copy(rows0, acc.at[idx_v.at[1, UT]], add=True)

        plsc.subcore_barrier()

        pltpu.sync_copy(acc.at[pl.ds(zlo, ZR)],
                        out.at[cid, pl.ds(zlo, ZR), pl.ds(0, d)])

    return pl.kernel(
        body,
        out_type=[jax.ShapeDtypeStruct((NC, N, 128), jnp.float32)],
        mesh=mesh,
        scratch_types=[
            pltpu.VMEM((2, UT + 1, 256), jnp.int32),
            pltpu.VMEM((256, d), jnp.float32),
            pltpu.VMEM((256, d), jnp.float32),
            pltpu.VMEM((256, d), jnp.float32),
            pltpu.VMEM_SHARED((N, d), jnp.float32),
            pltpu.SemaphoreType.DMA,
            pltpu.SemaphoreType.DMA,
            pltpu.SemaphoreType.DMA,
        ],
        compiler_params=pltpu.CompilerParams(use_tc_tiling_on_sc=False))


def _make_edge_scatter_512(d):
    nu5 = E // 512
    ut5 = nu5 // NW
    rem = nu5 - ut5 * NW
    nt = ut5 // 3
    mesh = plsc.VectorSubcoreMesh(core_axis_name="c", subcore_axis_name="s",
                                  num_cores=NC, num_subcores=NS)

    def body(table, e5d, zeros_d, out, idx_v, rows0, rows1, rows2, acc,
             sem0, sem1, sem2):
        cid = lax.axis_index("c")
        sid = lax.axis_index("s")
        tile = cid * NS + sid
        extra = tile < rem
        base = ut5 * tile + jnp.minimum(tile, rem)

        pltpu.sync_copy(e5d.at[:, pl.ds(base, ut5)],
                        idx_v.at[:, pl.ds(0, ut5)])

        @pl.when(extra)
        def _():
            pltpu.sync_copy(e5d.at[:, pl.ds(base + ut5, 1)],
                            idx_v.at[:, pl.ds(ut5, 1)])

        pltpu.async_copy(table.at[idx_v.at[0, 0]], rows0, sem0)
        pltpu.async_copy(table.at[idx_v.at[0, 1]], rows1, sem1)

        zlo = sid * ZR
        pltpu.sync_copy(zeros_d.at[pl.ds(zlo, ZR)], acc.at[pl.ds(zlo, ZR)])
        plsc.subcore_barrier()

        def triple(k, carry):
            u = 3 * k
            pltpu.async_copy(table.at[idx_v.at[0, u + 2]], rows2, sem2)
            pltpu.make_async_copy(table.at[idx_v.at[0, u]], rows0,
                                  sem0).wait()
            pltpu.sync_copy(rows0, acc.at[idx_v.at[1, u]], add=True)

            pltpu.async_copy(table.at[idx_v.at[0, u + 3]], rows0, sem0)

            pltpu.make_async_copy(table.at[idx_v.at[0, u + 1]], rows1,
                                  sem1).wait()
            pltpu.sync_copy(rows1, acc.at[idx_v.at[1, u + 1]], add=True)

            @pl.when((k < nt - 1) | extra)
            def _():
                pltpu.async_copy(table.at[idx_v.at[0, u + 4]], rows1, sem1)

            pltpu.make_async_copy(table.at[idx_v.at[0, u + 2]], rows2,
                                  sem2).wait()
            pltpu.sync_copy(rows2, acc.at[idx_v.at[1, u + 2]], add=True)
            return carry

        lax.fori_loop(0, nt, triple, None)

        pltpu.make_async_copy(table.at[idx_v.at[0, ut5 - 1]], rows0,
                              sem0).wait()
        pltpu.sync_copy(rows0, acc.at[idx_v.at[1, ut5 - 1]], add=True)

        @pl.when(extra)
        def _():
            pltpu.make_async_copy(table.at[idx_v.at[0, ut5]], rows1,
                                  sem1).wait()
            pltpu.sync_copy(rows1, acc.at[idx_v.at[1, ut5]], add=True)

        plsc.subcore_barrier()
        pltpu.sync_copy(acc.at[pl.ds(zlo, ZR)],
                        out.at[cid, pl.ds(zlo, ZR), pl.ds(0, d)])

    return pl.kernel(
        body,
        out_type=[jax.ShapeDtypeStruct((NC, N, 128), jnp.float32)],
        mesh=mesh,
        scratch_types=[
            pltpu.VMEM((2, ut5 + 1, 512), jnp.int32),
            pltpu.VMEM((512, d), jnp.float32),
            pltpu.VMEM((512, d), jnp.float32),
            pltpu.VMEM((512, d), jnp.float32),
            pltpu.VMEM_SHARED((N, d), jnp.float32),
            pltpu.SemaphoreType.DMA,
            pltpu.SemaphoreType.DMA,
            pltpu.SemaphoreType.DMA,
        ],
        compiler_params=pltpu.CompilerParams(use_tc_tiling_on_sc=False))


_edge_scatter_l1 = _make_edge_scatter(W1)
_edge_scatter_l2 = _make_edge_scatter_512(O)


_ZEROS_1 = np.zeros((N, W1), np.float32)
_ZEROS_2 = np.zeros((N, O), np.float32)


def kernel(x, edge_index, W1l, b1, W1r, W2l, b2, W2r):
    e3d = edge_index.astype(jnp.int32).reshape(2, NU, 256)
    zeros_1 = _ZEROS_1
    zeros_2 = _ZEROS_2

    p1 = _project_p1(x, W1l.T)
    part1, = _edge_scatter_l1(p1, e3d, zeros_1)
    r1 = _project_root(x, W1r.T, b1, F, H)
    p2, h, invc = _combine_project(part1, r1, W2l.T)
    part2, = _edge_scatter_l2(p2, e3d.reshape(2, E // 512, 512), zeros_2)
    r2 = _project_root(h, W2r.T, b2, H, O)
    return _final_combine(part2, invc, r2)
